# Initial kernel scaffold; baseline (speedup 1.0000x reference)
import jax
import jax.numpy as jnp
from jax.experimental import pallas as pl


def kernel(user_id, table):
    raise NotImplementedError("write your pallas kernel here")


# speedup vs baseline: 45.0744x; 45.0744x over previous
import functools

import jax
import jax.numpy as jnp
from jax import lax
from jax.experimental import pallas as pl
from jax.experimental.pallas import tpu as pltpu

_MIB = 1024 * 1024

_IDS_PER_STEP = 512
_UNROLL = 16


def _round_up(x: int, m: int) -> int:
    return ((x + m - 1) // m) * m


def _row_gather_kernel(ids_ref, table_ref, out_ref, *, ips, unroll):
    base = pl.program_id(0) * ips

    def outer(j, carry):
        k0 = j * unroll
        for u in range(unroll):
            k = k0 + u
            idx = ids_ref[base + k]
            out_ref[k, 0] = table_ref[idx, 0]
        return carry

    lax.fori_loop(0, ips // unroll, outer, 0)


def kernel(user_id: jax.Array, table: jax.Array) -> jax.Array:
    users_num, hidden = table.shape
    orig_shape = user_id.shape
    dtype = table.dtype

    flat_ids = user_id.reshape(-1).astype(jnp.int32)
    num_ids = flat_ids.shape[0]
    flat_ids = jnp.clip(flat_ids, 0, users_num - 1)

    hidden_p = _round_up(hidden, 128)
    table_p = table
    if hidden_p != hidden:
        table_p = jnp.pad(table, ((0, 0), (0, hidden_p - hidden)))
    table_3d = table_p.reshape(users_num, 1, hidden_p)

    ips = min(_IDS_PER_STEP, _round_up(num_ids, _UNROLL))
    num_steps = pl.cdiv(num_ids, ips)
    padded = num_steps * ips
    if padded != num_ids:
        flat_ids = jnp.pad(flat_ids, (0, padded - num_ids))

    out_shape = jax.ShapeDtypeStruct((padded, 1, hidden_p), dtype)
    itemsize = jnp.dtype(dtype).itemsize
    table_bytes = users_num * hidden_p * itemsize
    vmem_limit = int(min(56 * _MIB,
                         2 * table_bytes + 4 * ips * hidden_p * itemsize
                         + 8 * _MIB))
    compiler_params = pltpu.CompilerParams(
        dimension_semantics=("parallel",),
        vmem_limit_bytes=vmem_limit)
    body = functools.partial(_row_gather_kernel, ips=ips, unroll=_UNROLL)

    def build(single_buffer_table: bool):
        table_kwargs = {}
        if single_buffer_table:
            table_kwargs["pipeline_mode"] = pl.Buffered(1)
        grid_spec = pltpu.PrefetchScalarGridSpec(
            num_scalar_prefetch=1,
            grid=(num_steps,),
            in_specs=[
                pl.BlockSpec((users_num, 1, hidden_p),
                             lambda i, ids: (0, 0, 0), **table_kwargs),
            ],
            out_specs=pl.BlockSpec((ips, 1, hidden_p),
                                   lambda i, ids: (i, 0, 0)),
        )
        return pl.pallas_call(body, grid_spec=grid_spec,
                              out_shape=out_shape,
                              compiler_params=compiler_params)

    try:
        out = build(single_buffer_table=True)(flat_ids, table_3d)
    except Exception:
        out = build(single_buffer_table=False)(flat_ids, table_3d)

    out = out[:num_ids, 0, :hidden]
    return out.reshape(orig_shape + (hidden,))


# speedup vs baseline: 52.9553x; 1.1748x over previous
import functools

import jax
import jax.numpy as jnp
from jax import lax
from jax.experimental import pallas as pl
from jax.experimental.pallas import tpu as pltpu

_MIB = 1024 * 1024

_IDS_PER_STEP = 512
_UNROLL = 64


def _round_up(x: int, m: int) -> int:
    return ((x + m - 1) // m) * m


def _row_gather_kernel(ids_ref, table_ref, out_ref, *, ips, unroll):
    del unroll
    base = pl.program_id(0) * ips

    for k in range(ips):
        idx = ids_ref[base + k]
        out_ref[k, 0] = table_ref[idx, 0]


def kernel(user_id: jax.Array, table: jax.Array) -> jax.Array:
    users_num, hidden = table.shape
    orig_shape = user_id.shape
    dtype = table.dtype

    flat_ids = user_id.reshape(-1).astype(jnp.int32)
    num_ids = flat_ids.shape[0]
    flat_ids = jnp.clip(flat_ids, 0, users_num - 1)

    hidden_p = _round_up(hidden, 128)
    table_p = table
    if hidden_p != hidden:
        table_p = jnp.pad(table, ((0, 0), (0, hidden_p - hidden)))
    table_3d = table_p.reshape(users_num, 1, hidden_p)

    ips = min(_IDS_PER_STEP, _round_up(num_ids, _UNROLL))
    num_steps = pl.cdiv(num_ids, ips)
    padded = num_steps * ips
    if padded != num_ids:
        flat_ids = jnp.pad(flat_ids, (0, padded - num_ids))

    out_shape = jax.ShapeDtypeStruct((padded, 1, hidden_p), dtype)
    itemsize = jnp.dtype(dtype).itemsize
    table_bytes = users_num * hidden_p * itemsize
    vmem_limit = int(min(56 * _MIB,
                         2 * table_bytes + 4 * ips * hidden_p * itemsize
                         + 8 * _MIB))
    compiler_params = pltpu.CompilerParams(
        dimension_semantics=("parallel",),
        vmem_limit_bytes=vmem_limit)
    body = functools.partial(_row_gather_kernel, ips=ips, unroll=_UNROLL)

    def build(single_buffer_table: bool):
        table_kwargs = {}
        if single_buffer_table:
            table_kwargs["pipeline_mode"] = pl.Buffered(1)
        grid_spec = pltpu.PrefetchScalarGridSpec(
            num_scalar_prefetch=1,
            grid=(num_steps,),
            in_specs=[
                pl.BlockSpec((users_num, 1, hidden_p),
                             lambda i, ids: (0, 0, 0), **table_kwargs),
            ],
            out_specs=pl.BlockSpec((ips, 1, hidden_p),
                                   lambda i, ids: (i, 0, 0)),
        )
        return pl.pallas_call(body, grid_spec=grid_spec,
                              out_shape=out_shape,
                              compiler_params=compiler_params)

    try:
        out = build(single_buffer_table=True)(flat_ids, table_3d)
    except Exception:
        out = build(single_buffer_table=False)(flat_ids, table_3d)

    out = out[:num_ids, 0, :hidden]
    return out.reshape(orig_shape + (hidden,))


# speedup vs baseline: 52.9798x; 1.0005x over previous
import functools

import jax
import jax.numpy as jnp
from jax import lax
from jax.experimental import pallas as pl
from jax.experimental.pallas import tpu as pltpu

_MIB = 1024 * 1024

_IDS_PER_STEP = 512
_UNROLL = 64


def _round_up(x: int, m: int) -> int:
    return ((x + m - 1) // m) * m


def _row_gather_kernel(ids_ref, table_ref, out_ref, *, ips, unroll):
    del unroll
    base = pl.program_id(0) * ips

    for k in range(ips):
        idx = ids_ref[base + k]
        out_ref[k, 0] = table_ref[idx, 0]


def kernel(user_id: jax.Array, table: jax.Array) -> jax.Array:
    users_num, hidden = table.shape
    orig_shape = user_id.shape
    dtype = table.dtype

    flat_ids = user_id.reshape(-1).astype(jnp.int32)
    num_ids = flat_ids.shape[0]
    flat_ids = jnp.clip(flat_ids, 0, users_num - 1)

    hidden_p = _round_up(hidden, 128)
    table_p = table
    if hidden_p != hidden:
        table_p = jnp.pad(table, ((0, 0), (0, hidden_p - hidden)))
    table_3d = table_p.reshape(users_num, 1, hidden_p)

    ips = min(_IDS_PER_STEP, _round_up(num_ids, _UNROLL))
    num_steps = pl.cdiv(num_ids, ips)
    padded = num_steps * ips
    if padded != num_ids:
        flat_ids = jnp.pad(flat_ids, (0, padded - num_ids))

    out_shape = jax.ShapeDtypeStruct((padded, 1, hidden_p), dtype)
    itemsize = jnp.dtype(dtype).itemsize
    table_bytes = users_num * hidden_p * itemsize
    vmem_limit = int(min(56 * _MIB,
                         2 * table_bytes + 4 * ips * hidden_p * itemsize
                         + 8 * _MIB))
    compiler_params = pltpu.CompilerParams(
        dimension_semantics=("arbitrary",),
        vmem_limit_bytes=vmem_limit)
    body = functools.partial(_row_gather_kernel, ips=ips, unroll=_UNROLL)

    def build(single_buffer_table: bool):
        table_kwargs = {}
        if single_buffer_table:
            table_kwargs["pipeline_mode"] = pl.Buffered(1)
        grid_spec = pltpu.PrefetchScalarGridSpec(
            num_scalar_prefetch=1,
            grid=(num_steps,),
            in_specs=[
                pl.BlockSpec((users_num, 1, hidden_p),
                             lambda i, ids: (0, 0, 0), **table_kwargs),
            ],
            out_specs=pl.BlockSpec((ips, 1, hidden_p),
                                   lambda i, ids: (i, 0, 0)),
        )
        return pl.pallas_call(body, grid_spec=grid_spec,
                              out_shape=out_shape,
                              compiler_params=compiler_params)

    try:
        out = build(single_buffer_table=True)(flat_ids, table_3d)
    except Exception:
        out = build(single_buffer_table=False)(flat_ids, table_3d)

    out = out[:num_ids, 0, :hidden]
    return out.reshape(orig_shape + (hidden,))
